# baseline (speedup 1.0000x reference)
import functools

import jax
import jax.numpy as jnp
from jax import lax
from jax.experimental import pallas as pl
from jax.experimental.pallas import tpu as pltpu
from jax.experimental.pallas import tpu_sc as plsc

_LANES = 16


def _make_sc_kernel(N, HID, D, NC, NS, VT):
    NW = NC * NS
    rows_pw = N // NW
    C = _LANES
    n_chunks = rows_pw // C
    DP = D // 2
    mesh = plsc.VectorSubcoreMesh(core_axis_name="c", subcore_axis_name="s")

    @functools.partial(
        pl.kernel,
        mesh=mesh,
        compiler_params=pltpu.CompilerParams(needs_layout_passes=False),
        out_type=jax.ShapeDtypeStruct((N * HID,), jnp.float32),
        scratch_types=[
            pltpu.VMEM((4, rows_pw), jnp.int32),
            pltpu.VMEM((2 * C * HID,), jnp.float32),
            pltpu.VMEM((VT, DP), jnp.int32),
            pltpu.SemaphoreType.DMA,
            pltpu.SemaphoreType.DMA,
            pltpu.SemaphoreType.DMA,
            pltpu.SemaphoreType.DMA,
        ],
    )
    def k(x_hbm, idx_hbm, tab_hbm, out_hbm, idx_v, x_v, tab_v, si0, si1, so0, so1):
        s_in = (si0, si1)
        s_out = (so0, so1)
        wid = lax.axis_index("s") * NC + lax.axis_index("c")
        base = wid * rows_pw
        pltpu.sync_copy(tab_hbm, tab_v)
        pltpu.sync_copy(idx_hbm.at[:, pl.ds(base, rows_pw)], idx_v)

        def in_copy(g, s, es):
            return pltpu.make_async_copy(
                x_hbm.at[pl.ds((base + g * C) * HID, C * HID)],
                x_v.at[pl.ds(s * (C * HID), C * HID)],
                s_in[es],
            )

        def out_copy(g, s, es):
            return pltpu.make_async_copy(
                x_v.at[pl.ds(s * (C * HID), C * HID)],
                out_hbm.at[pl.ds((base + g * C) * HID, C * HID)],
                s_out[es],
            )

        def add_chunk(g, s):
            lanes = lax.iota(jnp.int32, _LANES)
            for j in range(4):
                rowvec = idx_v[j, pl.ds(g * C, _LANES)]
                tvec0 = lanes * 0
                wvec0 = lanes * HID + (s * (C * HID) + j * D)

                def pair(p, carry):
                    tvec, wvec = carry
                    g32 = plsc.load_gather(tab_v, [rowvec, tvec])
                    bb = plsc.bitcast(g32, jnp.bfloat16)
                    a, b = plsc.unpack(bb, format=plsc.PackFormat.INTERLEAVED)
                    plsc.addupdate_scatter(x_v, [wvec], a)
                    plsc.addupdate_scatter(x_v, [wvec + 1], b)
                    return (tvec + 1, wvec + 2)

                lax.fori_loop(0, DP, pair, (tvec0, wvec0), unroll=8)

        def super_chunk(g2, carry):
            for u in range(2):
                g = g2 * 2 + u

                @pl.when(g >= 1)
                def _drain_prev_out():
                    out_copy(g - 1, 1 - u, 1 - u).wait()

                @pl.when(g < n_chunks - 1)
                def _fire_next():
                    in_copy(g + 1, 1 - u, 1 - u).start()

                in_copy(g, u, u).wait()
                add_chunk(g, u)
                out_copy(g, u, u).start()
            return carry

        in_copy(0, 0, 0).start()
        lax.fori_loop(0, n_chunks // 2, super_chunk, 0, unroll=False)
        out_copy(n_chunks - 1, 1, 1).wait()

    return k


def kernel(x, chan_ind, spat_ind, embed_channel, embed_spatial):
    N, HID = x.shape
    VC, D = embed_channel.shape
    VS = embed_spatial.shape[0]
    idx_all = jnp.concatenate(
        [
            chan_ind.astype(jnp.int32),
            spat_ind.astype(jnp.int32) + VC,
        ],
        axis=1,
    ).T
    tab = jnp.concatenate([embed_channel, embed_spatial], axis=0)
    tab_packed = lax.bitcast_convert_type(
        tab.astype(jnp.bfloat16).reshape(VC + VS, D // 2, 2), jnp.int32
    )
    info = plsc.get_sparse_core_info()
    k = _make_sc_kernel(N, HID, D, info.num_cores, info.num_subcores, VC + VS)
    out = k(x.reshape(N * HID), idx_all, tab_packed)
    return out.reshape(N, HID)


# speedup vs baseline: 4.5403x; 4.5403x over previous
import functools

import jax
import jax.numpy as jnp
from jax import lax
from jax.experimental import pallas as pl
from jax.experimental.pallas import tpu as pltpu
from jax.experimental.pallas import tpu_sc as plsc

_LANES = 16


def _make_sc_kernel(N, HID, D, NC, NS, C):
    NW = NC * NS
    rows_pw = N // NW
    n_chunks = rows_pw // C
    idx_pw = 4 * rows_pw
    mesh = plsc.VectorSubcoreMesh(core_axis_name="c", subcore_axis_name="s")

    @functools.partial(
        pl.kernel,
        mesh=mesh,
        out_type=jax.ShapeDtypeStruct((N, HID), jnp.float32),
        scratch_types=[
            pltpu.VMEM((idx_pw,), jnp.int32),
            pltpu.VMEM((4, C, HID), jnp.float32),
            pltpu.VMEM((2, 4 * C, D), jnp.float32),
            pltpu.SemaphoreType.DMA,
            pltpu.SemaphoreType.DMA,
            pltpu.SemaphoreType.DMA,
            pltpu.SemaphoreType.DMA,
        ],
    )
    def k(x_hbm, idx_hbm, tab_hbm, out_hbm, idx_v, x_v, e_v, si0, si1, so0, so1):
        s_in = (si0, si1)
        s_out = (so0, so1)
        wid = lax.axis_index("s") * NC + lax.axis_index("c")
        base = wid * rows_pw
        pltpu.sync_copy(idx_hbm.at[pl.ds(wid * idx_pw, idx_pw)], idx_v)

        def in_copies(g, xs, es):
            r0 = base + g * C
            sem = s_in[es]
            return (
                pltpu.make_async_copy(x_hbm.at[pl.ds(r0, C), :], x_v.at[xs], sem),
                pltpu.make_async_copy(
                    tab_hbm.at[idx_v.at[pl.ds(g * (4 * C), 4 * C)]], e_v.at[es], sem
                ),
            )

        def out_copy(g, xs, es):
            r0 = base + g * C
            return pltpu.make_async_copy(
                x_v.at[xs], out_hbm.at[pl.ds(r0, C), :], s_out[es]
            )

        def add_chunk(xs, es):
            def row(c, carry):
                for j in range(4):
                    for t in range(D // _LANES):
                        plsc.addupdate(
                            x_v.at[xs, c, pl.ds(j * D + t * _LANES, _LANES)],
                            e_v[es, j * C + c, pl.ds(t * _LANES, _LANES)],
                        )
                return carry

            lax.fori_loop(0, C, row, 0, unroll=False)

        def super_chunk(g2, carry):
            for u in range(4):
                g = g2 * 4 + u
                b = u % 2

                @pl.when(g < n_chunks - 1)
                def _fire_next():
                    for d in in_copies(g + 1, (u + 1) % 4, 1 - b):
                        d.start()

                for d in in_copies(g, u, b):
                    d.wait()
                add_chunk(u, b)

                @pl.when(g >= 1)
                def _drain_prev_out():
                    out_copy(g - 1, (u + 3) % 4, 1 - b).wait()

                out_copy(g, u, b).start()
            return carry

        for d in in_copies(0, 0, 0):
            d.start()
        lax.fori_loop(0, n_chunks // 4, super_chunk, 0, unroll=False)
        out_copy(n_chunks - 1, 3, 1).wait()

    return k


def kernel(x, chan_ind, spat_ind, embed_channel, embed_spatial):
    N, HID = x.shape
    VC, D = embed_channel.shape
    C = 16
    tab = jnp.concatenate([embed_channel, embed_spatial], axis=0)
    idx_all = jnp.concatenate(
        [chan_ind.astype(jnp.int32), spat_ind.astype(jnp.int32) + VC], axis=1
    ).T
    idx_f = (
        idx_all.reshape(4, N // C, C).transpose(1, 0, 2).reshape(N * 4)
    )
    info = plsc.get_sparse_core_info()
    k = _make_sc_kernel(N, HID, D, info.num_cores, info.num_subcores, C)
    return k(x, idx_f, tab)
